# baseline (speedup 1.0000x reference)
import functools

import jax
import jax.numpy as jnp
from jax import lax
from jax.experimental import pallas as pl
from jax.experimental.pallas import tpu as pltpu
from jax.experimental.pallas import tpu_sc as plsc

VOCAB = 1000000
EMBED_DIM = 64
SKIP_DIM = 16
OUT_DIM = EMBED_DIM + SKIP_DIM

NC = 2
NS = 16
NW = NC * NS

B = 4096
L = 200
LT = L // 8
BPW = B // NW
NBUF = 2


def _body(x4, s4, item_hbm, skip_hbm, out_hbm,
          rawx, rawst, item_bufs, plane_bufs, skip_vm,
          sem_g, sem_w):
    wid = lax.axis_index("s") * NC + lax.axis_index("c")

    pltpu.sync_copy(x4.at[:, wid], rawx)
    pltpu.sync_copy(s4.at[:, wid], rawst)
    pltpu.sync_copy(skip_hbm, skip_vm.at[pl.ds(8, 2 * SKIP_DIM)])

    iota = lax.iota(jnp.int32, 16)
    bvecs = [iota + (bg * 16) for bg in range(8)]

    def fire_gather(l, p):
        lt = lax.div(l, 8)
        li = lax.rem(l, 8)
        return pltpu.async_copy(
            item_hbm.at[rawx.at[lt, li]], item_bufs[p], sem_g[p])

    def fire_write(l, p):
        return pltpu.async_copy(
            plane_bufs[p], out_hbm.at[l, :, wid], sem_w[p])

    for p in range(NBUF):
        fire_gather(p, p)

    def super_body(i, carry):
        for p in range(NBUF):
            l = i * NBUF + p
            lt = lax.div(l, 8)
            li = lax.rem(l, 8)
            pltpu.make_async_copy(
                item_hbm.at[rawx.at[lt, li]], item_bufs[p], sem_g[p]).wait()
            @pl.when(i >= 1)
            def _wait_prev():
                pltpu.make_async_copy(
                    plane_bufs[p], out_hbm.at[l - NBUF, :, wid],
                    sem_w[p]).wait()

            item = item_bufs[p]
            plane = plane_bufs[p]
            for dt in range(EMBED_DIM // 8):
                for din in range(8):
                    d = dt * 8 + din
                    dsplat = jnp.full((16,), d, jnp.int32)
                    for bg in range(8):
                        v = plsc.load_gather(item, [bvecs[bg], dsplat])
                        plane[dt, din, pl.ds(bg * 16, 16)] = v
            masks = [rawst[lt, li, pl.ds(bg * 16, 16)] != 0
                     for bg in range(8)]
            for ds in range(SKIP_DIM):
                dsplat = jnp.full((16,), 8 + ds, jnp.int32)
                t0 = plsc.load_gather(skip_vm, [dsplat])
                t1 = plsc.load_gather(skip_vm, [dsplat + SKIP_DIM])
                dt = EMBED_DIM // 8 + ds // 8
                din = ds % 8
                for bg in range(8):
                    plane[dt, din, pl.ds(bg * 16, 16)] = jnp.where(
                        masks[bg], t1, t0)

            @pl.when(i < (L // NBUF) - 1)
            def _prefetch():
                fire_gather(l + NBUF, p)

            fire_write(l, p)
        return carry

    lax.fori_loop(0, L // NBUF, super_body, 0)

    for p in range(NBUF):
        l = L - NBUF + p
        pltpu.make_async_copy(
            plane_bufs[p], out_hbm.at[l, :, wid], sem_w[p]).wait()


def _entry(x4, s4, item_hbm, skip_hbm, out_hbm, *scratch):
    rawx = scratch[0]
    rawst = scratch[1]
    item_bufs = scratch[2:2 + NBUF]
    plane_bufs = scratch[2 + NBUF:2 + 2 * NBUF]
    skip_vm = scratch[2 + 2 * NBUF]
    sem_g = scratch[3 + 2 * NBUF:3 + 3 * NBUF]
    sem_w = scratch[3 + 3 * NBUF:3 + 4 * NBUF]
    _body(x4, s4, item_hbm, skip_hbm, out_hbm,
          rawx, rawst, item_bufs, plane_bufs, skip_vm, sem_g, sem_w)


def kernel(x, skip_status, item_table, skip_table):
    x4 = x.reshape(NW, BPW, LT, 8).transpose(2, 0, 3, 1)
    s4 = skip_status.reshape(NW, BPW, LT, 8).transpose(2, 0, 3, 1)

    mesh = plsc.VectorSubcoreMesh(core_axis_name="c", subcore_axis_name="s")
    scratch = (
        [pltpu.VMEM((LT, 8, BPW), jnp.int32),
         pltpu.VMEM((LT, 8, BPW), jnp.int32)] +
        [pltpu.VMEM((BPW, EMBED_DIM), jnp.float32) for _ in range(NBUF)] +
        [pltpu.VMEM((OUT_DIM // 8, 8, BPW), jnp.float32)
         for _ in range(NBUF)] +
        [pltpu.VMEM((8 + 2 * SKIP_DIM,), jnp.float32)] +
        [pltpu.SemaphoreType.DMA for _ in range(2 * NBUF)]
    )
    run = functools.partial(
        pl.kernel,
        out_type=jax.ShapeDtypeStruct((L, OUT_DIM // 8, NW, 8, BPW),
                                      jnp.float32),
        mesh=mesh,
        scratch_types=scratch,
        compiler_params=pltpu.CompilerParams(use_tc_tiling_on_sc=False,
                                             needs_layout_passes=False),
    )(_entry)
    out5 = run(x4, s4, item_table, skip_table.reshape(2 * SKIP_DIM))
    return out5.transpose(2, 4, 0, 1, 3).reshape(B, L, OUT_DIM)


# speedup vs baseline: 1.2571x; 1.2571x over previous
import functools

import jax
import jax.numpy as jnp
from jax import lax
from jax.experimental import pallas as pl
from jax.experimental.pallas import tpu as pltpu
from jax.experimental.pallas import tpu_sc as plsc

VOCAB = 1000000
EMBED_DIM = 64
SKIP_DIM = 16
OUT_DIM = EMBED_DIM + SKIP_DIM

NC = 2
NS = 16
NW = NC * NS

B = 4096
L = 200
LT = L // 8
BPW = B // NW
NBUF = 2


def _body(x4, s4, item_hbm, skip_hbm, out_hbm,
          rawx, rawst, item_bufs, plane_bufs, skip_vm,
          sem_g, sem_w):
    wid = lax.axis_index("s") * NC + lax.axis_index("c")

    pltpu.sync_copy(x4.at[:, wid], rawx)
    pltpu.sync_copy(s4.at[:, wid], rawst)
    pltpu.sync_copy(skip_hbm, skip_vm.at[pl.ds(8, 2 * SKIP_DIM)])

    iota = lax.iota(jnp.int32, 16)
    bvecs = [iota + (bg * 16) for bg in range(8)]

    def fire_gather(l, p):
        lt = lax.div(l, 8)
        li = lax.rem(l, 8)
        return pltpu.async_copy(
            item_hbm.at[rawx.at[lt, li]], item_bufs[p], sem_g[p])

    def fire_write(l, p):
        return pltpu.async_copy(
            plane_bufs[p], out_hbm.at[l, :, wid], sem_w[p])

    for p in range(NBUF):
        fire_gather(p, p)

    def super_body(i, carry):
        for p in range(NBUF):
            l = i * NBUF + p
            lt = lax.div(l, 8)
            li = lax.rem(l, 8)
            pltpu.make_async_copy(
                item_hbm.at[rawx.at[lt, li]], item_bufs[p], sem_g[p]).wait()
            @pl.when(i >= 1)
            def _wait_prev():
                pltpu.make_async_copy(
                    plane_bufs[p], out_hbm.at[l - NBUF, :, wid],
                    sem_w[p]).wait()

            item = item_bufs[p]
            plane = plane_bufs[p]
            for dt in range(EMBED_DIM // 8):
                for din in range(8):
                    d = dt * 8 + din
                    dsplat = jnp.full((16,), d, jnp.int32)
                    vs = [plsc.load_gather(item, [bvecs[bg], dsplat])
                          for bg in range(8)]
                    for bg in range(8):
                        plane[dt, din, pl.ds(bg * 16, 16)] = vs[bg]
            masks = [rawst[lt, li, pl.ds(bg * 16, 16)] != 0
                     for bg in range(8)]
            tsplats = []
            for ds in range(SKIP_DIM):
                dsplat = jnp.full((16,), 8 + ds, jnp.int32)
                tsplats.append((plsc.load_gather(skip_vm, [dsplat]),
                                plsc.load_gather(skip_vm,
                                                 [dsplat + SKIP_DIM])))
            for ds in range(SKIP_DIM):
                t0, t1 = tsplats[ds]
                dt = EMBED_DIM // 8 + ds // 8
                din = ds % 8
                sels = [jnp.where(masks[bg], t1, t0) for bg in range(8)]
                for bg in range(8):
                    plane[dt, din, pl.ds(bg * 16, 16)] = sels[bg]

            @pl.when(i < (L // NBUF) - 1)
            def _prefetch():
                fire_gather(l + NBUF, p)

            fire_write(l, p)
        return carry

    lax.fori_loop(0, L // NBUF, super_body, 0)

    for p in range(NBUF):
        l = L - NBUF + p
        pltpu.make_async_copy(
            plane_bufs[p], out_hbm.at[l, :, wid], sem_w[p]).wait()


def _entry(x4, s4, item_hbm, skip_hbm, out_hbm, *scratch):
    rawx = scratch[0]
    rawst = scratch[1]
    item_bufs = scratch[2:2 + NBUF]
    plane_bufs = scratch[2 + NBUF:2 + 2 * NBUF]
    skip_vm = scratch[2 + 2 * NBUF]
    sem_g = scratch[3 + 2 * NBUF:3 + 3 * NBUF]
    sem_w = scratch[3 + 3 * NBUF:3 + 4 * NBUF]
    _body(x4, s4, item_hbm, skip_hbm, out_hbm,
          rawx, rawst, item_bufs, plane_bufs, skip_vm, sem_g, sem_w)


def kernel(x, skip_status, item_table, skip_table):
    x4 = x.reshape(NW, BPW, LT, 8).transpose(2, 0, 3, 1)
    s4 = skip_status.reshape(NW, BPW, LT, 8).transpose(2, 0, 3, 1)

    mesh = plsc.VectorSubcoreMesh(core_axis_name="c", subcore_axis_name="s")
    scratch = (
        [pltpu.VMEM((LT, 8, BPW), jnp.int32),
         pltpu.VMEM((LT, 8, BPW), jnp.int32)] +
        [pltpu.VMEM((BPW, EMBED_DIM), jnp.float32) for _ in range(NBUF)] +
        [pltpu.VMEM((OUT_DIM // 8, 8, BPW), jnp.float32)
         for _ in range(NBUF)] +
        [pltpu.VMEM((8 + 2 * SKIP_DIM,), jnp.float32)] +
        [pltpu.SemaphoreType.DMA for _ in range(2 * NBUF)]
    )
    run = functools.partial(
        pl.kernel,
        out_type=jax.ShapeDtypeStruct((L, OUT_DIM // 8, NW, 8, BPW),
                                      jnp.float32),
        mesh=mesh,
        scratch_types=scratch,
        compiler_params=pltpu.CompilerParams(use_tc_tiling_on_sc=False,
                                             needs_layout_passes=False),
    )(_entry)
    out5 = run(x4, s4, item_table, skip_table.reshape(2 * SKIP_DIM))
    return out5.transpose(2, 4, 0, 1, 3).reshape(B, L, OUT_DIM)


# speedup vs baseline: 2.5123x; 1.9985x over previous
import functools

import jax
import jax.numpy as jnp
from jax import lax
from jax.experimental import pallas as pl
from jax.experimental.pallas import tpu as pltpu
from jax.experimental.pallas import tpu_sc as plsc

VOCAB = 1000000
EMBED_DIM = 64
SKIP_DIM = 16
OUT_DIM = EMBED_DIM + SKIP_DIM

NC = 2
NS = 16
NW = NC * NS

B = 4096
L = 200
LT = L // 8
BPW = B // NW
NBUF = 2


def _body(x4, s4, item_hbm, skip_hbm, out_hbm,
          rawx, rawst, item_bufs, plane_bufs, skip_vm,
          sem_g, sem_w):
    wid = lax.axis_index("s") * NC + lax.axis_index("c")

    pltpu.sync_copy(x4.at[:, wid], rawx)
    pltpu.sync_copy(s4.at[:, wid], rawst)
    pltpu.sync_copy(skip_hbm, skip_vm.at[pl.ds(8, 2 * SKIP_DIM)])

    iota = lax.iota(jnp.int32, 16)
    eight = jnp.full((16,), 8, jnp.int32)
    dcols = [iota + (j * 16) for j in range(4)]
    dtvs = [lax.div(c, eight) for c in dcols]
    hvs = [lax.rem(c, eight) * 128 for c in dcols]

    def fire_gather(l, p):
        lt = lax.div(l, 8)
        li = lax.rem(l, 8)
        return pltpu.async_copy(
            item_hbm.at[rawx.at[lt, li]], item_bufs[p], sem_g[p])

    def fire_write(l, p):
        return pltpu.async_copy(
            plane_bufs[p], out_hbm.at[l, :, wid], sem_w[p])

    for p in range(NBUF):
        fire_gather(p, p)

    def super_body(i, carry):
        for p in range(NBUF):
            l = i * NBUF + p
            lt = lax.div(l, 8)
            li = lax.rem(l, 8)
            pltpu.make_async_copy(
                item_hbm.at[rawx.at[lt, li]], item_bufs[p], sem_g[p]).wait()
            @pl.when(i >= 1)
            def _wait_prev():
                pltpu.make_async_copy(
                    plane_bufs[p], out_hbm.at[l - NBUF, :, wid],
                    sem_w[p]).wait()

            item = item_bufs[p]
            plane = plane_bufs[p]
            def kbody(k, carry3):
                rowoff = lax.rem(iota + k, jnp.full((16,), 16, jnp.int32))
                for j in range(4):
                    rows = [rowoff + b0 for b0 in range(0, BPW, 16)]
                    vs = [plsc.load_gather(item, [r, dcols[j]])
                          for r in rows]
                    for m in range(8):
                        plsc.store_scatter(
                            plane, [dtvs[j], hvs[j] + rows[m]], vs[m])
                return carry3

            lax.fori_loop(0, 16, kbody, 0)
            masks = [rawst[lt, li, pl.ds(bg * 16, 16)] != 0
                     for bg in range(8)]
            for ds0 in range(0, SKIP_DIM, 4):
                tsplats = []
                for ds in range(ds0, ds0 + 4):
                    dsplat = jnp.full((16,), 8 + ds, jnp.int32)
                    tsplats.append((plsc.load_gather(skip_vm, [dsplat]),
                                    plsc.load_gather(skip_vm,
                                                     [dsplat + SKIP_DIM])))
                for ds in range(ds0, ds0 + 4):
                    t0, t1 = tsplats[ds - ds0]
                    dt = EMBED_DIM // 8 + ds // 8
                    din = ds % 8
                    sels = [jnp.where(masks[bg], t1, t0) for bg in range(8)]
                    for bg in range(8):
                        plane[dt, pl.ds(din * 128 + bg * 16, 16)] = sels[bg]

            @pl.when(i < (L // NBUF) - 1)
            def _prefetch():
                fire_gather(l + NBUF, p)

            fire_write(l, p)
        return carry

    lax.fori_loop(0, L // NBUF, super_body, 0)

    for p in range(NBUF):
        l = L - NBUF + p
        pltpu.make_async_copy(
            plane_bufs[p], out_hbm.at[l, :, wid], sem_w[p]).wait()


def _entry(x4, s4, item_hbm, skip_hbm, out_hbm, *scratch):
    rawx = scratch[0]
    rawst = scratch[1]
    item_bufs = scratch[2:2 + NBUF]
    plane_bufs = scratch[2 + NBUF:2 + 2 * NBUF]
    skip_vm = scratch[2 + 2 * NBUF]
    sem_g = scratch[3 + 2 * NBUF:3 + 3 * NBUF]
    sem_w = scratch[3 + 3 * NBUF:3 + 4 * NBUF]
    _body(x4, s4, item_hbm, skip_hbm, out_hbm,
          rawx, rawst, item_bufs, plane_bufs, skip_vm, sem_g, sem_w)


def kernel(x, skip_status, item_table, skip_table):
    x4 = x.reshape(NW, BPW, LT, 8).transpose(2, 0, 3, 1)
    s4 = skip_status.reshape(NW, BPW, LT, 8).transpose(2, 0, 3, 1)

    mesh = plsc.VectorSubcoreMesh(core_axis_name="c", subcore_axis_name="s")
    scratch = (
        [pltpu.VMEM((LT, 8, BPW), jnp.int32),
         pltpu.VMEM((LT, 8, BPW), jnp.int32)] +
        [pltpu.VMEM((BPW, EMBED_DIM), jnp.float32) for _ in range(NBUF)] +
        [pltpu.VMEM((OUT_DIM // 8, 8 * BPW), jnp.float32)
         for _ in range(NBUF)] +
        [pltpu.VMEM((8 + 2 * SKIP_DIM,), jnp.float32)] +
        [pltpu.SemaphoreType.DMA for _ in range(2 * NBUF)]
    )
    run = functools.partial(
        pl.kernel,
        out_type=jax.ShapeDtypeStruct((L, OUT_DIM // 8, NW, 8 * BPW),
                                      jnp.float32),
        mesh=mesh,
        scratch_types=scratch,
        compiler_params=pltpu.CompilerParams(use_tc_tiling_on_sc=False,
                                             needs_layout_passes=False),
    )(_entry)
    out5 = run(x4, s4, item_table, skip_table.reshape(2 * SKIP_DIM))
    out6 = out5.reshape(L, OUT_DIM // 8, NW, 8, BPW)
    return out6.transpose(2, 4, 0, 1, 3).reshape(B, L, OUT_DIM)
